# baseline (speedup 1.0000x reference)
import jax
import jax.numpy as jnp
from jax import lax
from jax.experimental import pallas as pl
from jax.experimental.pallas import tpu as pltpu
from jax.experimental.pallas import tpu_sc as plsc

N = 10000
E = 320000
D = 128
NG = 128
NCLS = 2

TRI = N * (N - 1) // 2
NC, NS = 2, 16
NSUB = NC * NS
ROWS = -(-E // (NSUB * 128))
CH = ROWS * 128
GP = NSUB * CH
TRASH = N
STRIPE = 632
GG = NS * STRIPE


def _wid():
    return lax.axis_index("s") * NC + lax.axis_index("c")


def _t0_codes(src_ref, dst_ref, code_ref, tick_ref):
    s = src_ref[...]
    d = dst_ref[...]
    lo = jnp.minimum(s, d)
    hi = jnp.maximum(s, d)
    tri = lo * N - (lo * (lo + 1)) // 2 + (hi - lo - 1)
    nrows = GP // 128
    t = (lax.broadcasted_iota(jnp.int32, (nrows, 128), 0) * 128
         + lax.broadcasted_iota(jnp.int32, (nrows, 128), 1))
    loop = s == d
    code_ref[...] = jnp.where(loop, TRI, tri)
    tick_ref[...] = jnp.where(loop, 0, t + 1)


def _s1_scatter(codes_f, ticks_f, slot, idx_v, val_v, sem):
    wid = _wid()
    base = wid * CH
    pltpu.sync_copy(codes_f.at[pl.ds(base, CH)], idx_v)
    pltpu.sync_copy(ticks_f.at[pl.ds(base, CH)], val_v)
    pltpu.async_copy(val_v, slot.at[idx_v], sem).wait()


def _s2_degree(codes_f, srcd, dstd, slot, parts, ga, gb, idx_v, src_v,
               dst_v, got_v, ga_v, gb_v, deg_v, sem):
    wid = _wid()
    base = wid * CH
    pltpu.sync_copy(codes_f.at[pl.ds(base, CH)], idx_v)
    pltpu.sync_copy(srcd.at[pl.ds(base, CH)], src_v)
    pltpu.sync_copy(dstd.at[pl.ds(base, CH)], dst_v)
    pltpu.async_copy(slot.at[idx_v], got_v, sem).wait()

    def zero(i, c):
        deg_v[pl.ds(i * 16, 16)] = jnp.zeros((16,), jnp.float32)
        return c

    lax.fori_loop(0, N // 16, zero, 0)

    def acc(r, c):
        for cc in range(8):
            off = r * 128 + cc * 16
            got = got_v[pl.ds(off, 16)]
            s = src_v[pl.ds(off, 16)]
            d = dst_v[pl.ds(off, 16)]
            gt = base + off + lax.iota(jnp.int32, 16)
            keepb = got == gt + 1
            keep = jnp.where(keepb, 1.0, 0.0)
            plsc.addupdate_scatter(deg_v, [s], keep)
            plsc.addupdate_scatter(deg_v, [d], keep)
            ga_v[pl.ds(off, 16)] = jnp.where(keepb, s, TRASH)
            gb_v[pl.ds(off, 16)] = jnp.where(keepb, d, TRASH)
        return c

    lax.fori_loop(0, ROWS, acc, 0)
    pltpu.sync_copy(deg_v, parts.at[wid])
    pltpu.sync_copy(ga_v, ga.at[pl.ds(base, CH)])
    pltpu.sync_copy(gb_v, gb.at[pl.ds(base, CH)])


def _t1_prep(parts_ref, x_ref, w1_ref, g_ref, dinv_ref):
    ones32 = jnp.ones((NSUB, 1), jnp.float32)
    deg_col = lax.dot_general(parts_ref[...], ones32,
                              (((0,), (0,)), ((), ())),
                              preferred_element_type=jnp.float32,
                              precision=lax.Precision.HIGHEST) + 1.0
    dinv_col = lax.rsqrt(deg_col)
    h = jnp.dot(x_ref[...], w1_ref[...], preferred_element_type=jnp.float32,
                precision=lax.Precision.HIGHEST)
    g_ref[...] = jnp.concatenate(
        [h * dinv_col, jnp.zeros((GG - N, D), jnp.float32)])
    dinv_ref[...] = dinv_col


def _s3_spmm(ga_f, gb_f, gpad, zrows, part_out, shared, rows0, rows1,
             gi0, si0, gi1, si1, semg, sems0, sems1, semi):
    cid = lax.axis_index("c")
    sid = lax.axis_index("s")
    wid = sid * NC + cid
    base = wid * CH
    pltpu.sync_copy(zrows, shared.at[pl.ds(sid * STRIPE, STRIPE)])
    plsc.subcore_barrier()

    rows = (rows0, rows1)
    gi = (gi0, gi1)
    si = (si0, si1)
    gsrc = (ga_f, gb_f)
    ssrc = (gb_f, ga_f)

    sems = (sems0, sems1)

    def load_and_gather(c, p):
        pltpu.async_copy(gsrc[p].at[pl.ds(base + c * 128, 128)], gi[p], semi)
        pltpu.async_copy(ssrc[p].at[pl.ds(base + c * 128, 128)], si[p], semi)
        pltpu.make_async_copy(gsrc[p].at[pl.ds(base, 128)], gi[p], semi).wait()
        pltpu.make_async_copy(gsrc[p].at[pl.ds(base, 128)], si[p], semi).wait()
        pltpu.async_copy(gpad.at[gi[p]], rows[p], semg)

    def start_scatter(p):
        pltpu.make_async_copy(gpad.at[gi[p]], rows[p], semg).wait()
        pltpu.async_copy(rows[p], shared.at[si[p]], sems[p], add=True)

    def drain_scatter(p):
        pltpu.make_async_copy(rows[p], shared.at[si[p]], sems[p]).wait()

    load_and_gather(0, 0)
    start_scatter(0)
    load_and_gather(0, 1)

    def body(i, carry):
        start_scatter(1)
        drain_scatter(0)
        load_and_gather(i + 1, 0)
        start_scatter(0)
        drain_scatter(1)
        load_and_gather(i + 1, 1)
        return carry

    lax.fori_loop(0, ROWS - 1, body, 0)
    start_scatter(1)
    drain_scatter(0)
    drain_scatter(1)
    plsc.subcore_barrier()
    pltpu.sync_copy(shared.at[pl.ds(sid * STRIPE, STRIPE)],
                    part_out.at[cid, pl.ds(sid * STRIPE, STRIPE)])


def _t3_head(p_ref, g_ref, dinv_ref, b1_ref, x_ref, batch_ref, shift_ref,
             w0_ref, b0_ref, wl1_ref, bl1_ref, w2_ref, b2_ref, out_ref):
    hi = lax.Precision.HIGHEST
    psum = p_ref[0, :N, :] + p_ref[1, :N, :] + g_ref[:N, :]
    h1 = jnp.maximum(psum * dinv_ref[...] + b1_ref[...], 0.0)

    batch_col = batch_ref[...]
    shift_col = shift_ref[...]
    gid_row = lax.broadcasted_iota(jnp.int32, (1, NG), 1)
    first_ge = (batch_col >= gid_row) & (shift_col < gid_row)
    node_col = lax.broadcasted_iota(jnp.int32, (N, 1), 0)
    overflow = (node_col == N - 1) & (batch_col < gid_row)
    onehot = jnp.where(first_ge | overflow, 1.0, 0.0)
    xr = lax.dot_general(onehot, x_ref[...], (((0,), (0,)), ((), ())),
                         preferred_element_type=jnp.float32, precision=hi)
    news = jnp.maximum(
        jnp.dot(xr, w0_ref[...], preferred_element_type=jnp.float32,
                precision=hi) + b0_ref[...], 0.0)

    gi_col = lax.broadcasted_iota(jnp.int32, (NG, 1), 0)

    def seg(g, hp):
        m2 = jnp.max(jnp.where(batch_col == g, h1, -jnp.inf), axis=0,
                     keepdims=True)
        return jnp.maximum(hp, jnp.where(gi_col == g, m2, -jnp.inf))

    hp = lax.fori_loop(0, NG, seg, jnp.full((NG, D), -jnp.inf,
                                            dtype=jnp.float32))
    cat = jnp.concatenate([news, hp], axis=1)
    h2 = jnp.maximum(
        jnp.dot(cat, wl1_ref[...], preferred_element_type=jnp.float32,
                precision=hi) + bl1_ref[...], 0.0)
    logits = jnp.dot(h2, w2_ref[...], preferred_element_type=jnp.float32,
                     precision=hi) + b2_ref[...]
    mx = jnp.max(logits, axis=1, keepdims=True)
    lse = mx + jnp.log(jnp.sum(jnp.exp(logits - mx), axis=1, keepdims=True))
    out_ref[...] = logits - lse


def kernel(x, edge_index, batch, W1, b1, W0, b0, Wl1, bl1, W2, b2):
    ei0 = edge_index[0].astype(jnp.int32)
    ei1 = edge_index[1].astype(jnp.int32)
    pad = jnp.zeros((GP - E,), jnp.int32)
    srcd = jnp.concatenate([ei0, pad])
    dstd = jnp.concatenate([ei1, pad])

    nrows_g = GP // 128
    codes_f, ticks_f = pl.pallas_call(
        _t0_codes,
        in_specs=[
            pl.BlockSpec((nrows_g, 128), lambda: (0, 0)),
            pl.BlockSpec((nrows_g, 128), lambda: (0, 0)),
        ],
        out_specs=[
            pl.BlockSpec((nrows_g, 128), lambda: (0, 0)),
            pl.BlockSpec((nrows_g, 128), lambda: (0, 0)),
        ],
        out_shape=[
            jax.ShapeDtypeStruct((nrows_g, 128), jnp.int32),
            jax.ShapeDtypeStruct((nrows_g, 128), jnp.int32),
        ],
    )(srcd.reshape(nrows_g, 128), dstd.reshape(nrows_g, 128))
    codes_flat = codes_f.reshape(GP)
    ticks_flat = ticks_f.reshape(GP)

    mesh = plsc.VectorSubcoreMesh(core_axis_name="c", subcore_axis_name="s")

    slot_ref = jax.new_ref(jnp.zeros((TRI + 8,), jnp.int32))
    pl.kernel(
        _s1_scatter,
        out_type=(),
        mesh=mesh,
        scratch_types=[
            pltpu.VMEM((CH,), jnp.int32),
            pltpu.VMEM((CH,), jnp.int32),
            pltpu.SemaphoreType.DMA,
        ],
    )(codes_flat, ticks_flat, slot_ref)
    slot = jax.freeze(slot_ref)

    parts, ga, gb = pl.kernel(
        _s2_degree,
        out_type=(
            jax.ShapeDtypeStruct((NSUB, N), jnp.float32),
            jax.ShapeDtypeStruct((GP,), jnp.int32),
            jax.ShapeDtypeStruct((GP,), jnp.int32),
        ),
        mesh=mesh,
        scratch_types=[
            pltpu.VMEM((CH,), jnp.int32),
            pltpu.VMEM((CH,), jnp.int32),
            pltpu.VMEM((CH,), jnp.int32),
            pltpu.VMEM((CH,), jnp.int32),
            pltpu.VMEM((CH,), jnp.int32),
            pltpu.VMEM((CH,), jnp.int32),
            pltpu.VMEM((N,), jnp.float32),
            pltpu.SemaphoreType.DMA,
        ],
        compiler_params=pltpu.CompilerParams(needs_layout_passes=False),
    )(codes_flat, srcd, dstd, slot)

    g, dinv = pl.pallas_call(
        _t1_prep,
        in_specs=[
            pl.BlockSpec((NSUB, N), lambda: (0, 0)),
            pl.BlockSpec((N, D), lambda: (0, 0)),
            pl.BlockSpec((D, D), lambda: (0, 0)),
        ],
        out_specs=[
            pl.BlockSpec((GG, D), lambda: (0, 0)),
            pl.BlockSpec((N, 1), lambda: (0, 0)),
        ],
        out_shape=[
            jax.ShapeDtypeStruct((GG, D), jnp.float32),
            jax.ShapeDtypeStruct((N, 1), jnp.float32),
        ],
    )(parts, x, W1)

    zrows = jnp.zeros((STRIPE, D), jnp.float32)
    part_out = pl.kernel(
        _s3_spmm,
        out_type=jax.ShapeDtypeStruct((NC, GG, D), jnp.float32),
        mesh=mesh,
        scratch_types=[
            pltpu.VMEM_SHARED((GG, D), jnp.float32),
            pltpu.VMEM((128, D), jnp.float32),
            pltpu.VMEM((128, D), jnp.float32),
            pltpu.VMEM((128,), jnp.int32),
            pltpu.VMEM((128,), jnp.int32),
            pltpu.VMEM((128,), jnp.int32),
            pltpu.VMEM((128,), jnp.int32),
            pltpu.SemaphoreType.DMA,
            pltpu.SemaphoreType.DMA,
            pltpu.SemaphoreType.DMA,
            pltpu.SemaphoreType.DMA,
        ],
    )(ga, gb, g, zrows)

    out = pl.pallas_call(
        _t3_head,
        in_specs=[
            pl.BlockSpec((NC, GG, D), lambda: (0, 0, 0)),
            pl.BlockSpec((GG, D), lambda: (0, 0)),
            pl.BlockSpec((N, 1), lambda: (0, 0)),
            pl.BlockSpec((1, D), lambda: (0, 0)),
            pl.BlockSpec((N, D), lambda: (0, 0)),
            pl.BlockSpec((N, 1), lambda: (0, 0)),
            pl.BlockSpec((N, 1), lambda: (0, 0)),
            pl.BlockSpec((D, D), lambda: (0, 0)),
            pl.BlockSpec((1, D), lambda: (0, 0)),
            pl.BlockSpec((2 * D, D), lambda: (0, 0)),
            pl.BlockSpec((1, D), lambda: (0, 0)),
            pl.BlockSpec((D, NCLS), lambda: (0, 0)),
            pl.BlockSpec((1, NCLS), lambda: (0, 0)),
        ],
        out_specs=pl.BlockSpec((NG, NCLS), lambda: (0, 0)),
        out_shape=jax.ShapeDtypeStruct((NG, NCLS), jnp.float32),
    )(part_out, g, dinv, b1.reshape(1, D), x,
      batch.astype(jnp.int32).reshape(N, 1),
      jnp.concatenate([jnp.full((1,), -1, jnp.int32),
                       batch.astype(jnp.int32)[:-1]]).reshape(N, 1),
      W0, b0.reshape(1, D), Wl1, bl1.reshape(1, D), W2, b2.reshape(1, NCLS))
    return out


# speedup vs baseline: 1.1759x; 1.1759x over previous
import jax
import jax.numpy as jnp
from jax import lax
from jax.experimental import pallas as pl
from jax.experimental.pallas import tpu as pltpu
from jax.experimental.pallas import tpu_sc as plsc

N = 10000
E = 320000
D = 128
NG = 128
NCLS = 2

TRI = N * (N - 1) // 2
NC, NS = 2, 16
NSUB = NC * NS
ROWS = -(-E // (NSUB * 128))
CH = ROWS * 128
GP = NSUB * CH
TRASH = N
STRIPE = 632
GG = NS * STRIPE
PSTR = 320
HP = NSUB * PSTR


def _wid():
    return lax.axis_index("s") * NC + lax.axis_index("c")


def _t0_codes(src_ref, dst_ref, code_ref, tick_ref):
    s = src_ref[...]
    d = dst_ref[...]
    lo = jnp.minimum(s, d)
    hi = jnp.maximum(s, d)
    tri = lo * N - (lo * (lo + 1)) // 2 + (hi - lo - 1)
    nrows = GP // 128
    t = (lax.broadcasted_iota(jnp.int32, (nrows, 128), 0) * 128
         + lax.broadcasted_iota(jnp.int32, (nrows, 128), 1))
    loop = s == d
    code_ref[...] = jnp.where(loop, TRI, tri)
    tick_ref[...] = jnp.where(loop, 0, t + 1)


def _s1_scatter(codes_f, ticks_f, slot, idx_v, val_v, sem):
    wid = _wid()
    base = wid * CH
    pltpu.sync_copy(codes_f.at[pl.ds(base, CH)], idx_v)
    pltpu.sync_copy(ticks_f.at[pl.ds(base, CH)], val_v)
    pltpu.async_copy(val_v, slot.at[idx_v], sem).wait()


def _s2_degree(codes_f, srcd, dstd, slot, parts, ga, gb, idx_v, src_v,
               dst_v, got_v, ga_v, gb_v, deg_v, sem):
    wid = _wid()
    base = wid * CH
    pltpu.sync_copy(codes_f.at[pl.ds(base, CH)], idx_v)
    pltpu.sync_copy(srcd.at[pl.ds(base, CH)], src_v)
    pltpu.sync_copy(dstd.at[pl.ds(base, CH)], dst_v)
    pltpu.async_copy(slot.at[idx_v], got_v, sem).wait()

    def zero(i, c):
        deg_v[pl.ds(i * 16, 16)] = jnp.zeros((16,), jnp.float32)
        return c

    lax.fori_loop(0, N // 16, zero, 0)

    def acc(r, c):
        for cc in range(8):
            off = r * 128 + cc * 16
            got = got_v[pl.ds(off, 16)]
            s = src_v[pl.ds(off, 16)]
            d = dst_v[pl.ds(off, 16)]
            gt = base + off + lax.iota(jnp.int32, 16)
            keepb = got == gt + 1
            keep = jnp.where(keepb, 1.0, 0.0)
            plsc.addupdate_scatter(deg_v, [s], keep)
            plsc.addupdate_scatter(deg_v, [d], keep)
            ga_v[pl.ds(off, 16)] = jnp.where(keepb, s, TRASH)
            gb_v[pl.ds(off, 16)] = jnp.where(keepb, d, TRASH)
        return c

    lax.fori_loop(0, ROWS, acc, 0)
    pltpu.sync_copy(deg_v, parts.at[wid])
    pltpu.sync_copy(ga_v, ga.at[pl.ds(base, CH)])
    pltpu.sync_copy(gb_v, gb.at[pl.ds(base, CH)])


def _t1_prep(parts_ref, x_ref, w1_ref, g_ref, dinv_ref):
    ones32 = jnp.ones((NSUB, 1), jnp.float32)
    deg_col = lax.dot_general(parts_ref[...], ones32,
                              (((0,), (0,)), ((), ())),
                              preferred_element_type=jnp.float32,
                              precision=lax.Precision.HIGHEST) + 1.0
    dinv_col = lax.rsqrt(deg_col)
    h = jnp.dot(x_ref[...], w1_ref[...], preferred_element_type=jnp.float32,
                precision=lax.Precision.HIGHEST)
    g_ref[...] = jnp.concatenate(
        [h * dinv_col, jnp.zeros((GG - N, D), jnp.float32)])
    dinv_ref[...] = dinv_col


def _s3_spmm(ga_f, gb_f, gpad, zrows, part_out, shared, rows0, rows1,
             gi0, si0, gi1, si1, semg, sems0, sems1, semi):
    cid = lax.axis_index("c")
    sid = lax.axis_index("s")
    wid = sid * NC + cid
    base = wid * CH
    pltpu.sync_copy(zrows, shared.at[pl.ds(sid * STRIPE, STRIPE)])
    plsc.subcore_barrier()

    rows = (rows0, rows1)
    gi = (gi0, gi1)
    si = (si0, si1)
    gsrc = (ga_f, gb_f)
    ssrc = (gb_f, ga_f)

    sems = (sems0, sems1)

    def load_and_gather(c, p):
        pltpu.async_copy(gsrc[p].at[pl.ds(base + c * 128, 128)], gi[p], semi)
        pltpu.async_copy(ssrc[p].at[pl.ds(base + c * 128, 128)], si[p], semi)
        pltpu.make_async_copy(gsrc[p].at[pl.ds(base, 128)], gi[p], semi).wait()
        pltpu.make_async_copy(gsrc[p].at[pl.ds(base, 128)], si[p], semi).wait()
        pltpu.async_copy(gpad.at[gi[p]], rows[p], semg)

    def start_scatter(p):
        pltpu.make_async_copy(gpad.at[gi[p]], rows[p], semg).wait()
        pltpu.async_copy(rows[p], shared.at[si[p]], sems[p], add=True)

    def drain_scatter(p):
        pltpu.make_async_copy(rows[p], shared.at[si[p]], sems[p]).wait()

    load_and_gather(0, 0)
    load_and_gather(0, 1)

    def body(i, carry):
        for p in (0, 1):
            start_scatter(p)
            drain_scatter(p)
            load_and_gather(i + 1, p)
        return carry

    lax.fori_loop(0, ROWS - 1, body, 0)
    for p in (0, 1):
        start_scatter(p)
        drain_scatter(p)
    plsc.subcore_barrier()
    pltpu.sync_copy(shared.at[pl.ds(sid * STRIPE, STRIPE)],
                    part_out.at[cid, pl.ds(sid * STRIPE, STRIPE)])


def _t3_h1(p_ref, g_ref, dinv_ref, b1_ref, h1p_ref):
    psum = p_ref[0, :N, :] + p_ref[1, :N, :] + g_ref[:N, :]
    h1 = jnp.maximum(psum * dinv_ref[...] + b1_ref[...], 0.0)
    h1p_ref[...] = jnp.concatenate(
        [h1, jnp.full((HP - N, D), -jnp.inf, jnp.float32)])


def _s4_pool(h1p, batchp, pools, hv, bv, hpl):
    wid = _wid()
    pltpu.sync_copy(h1p.at[pl.ds(wid * PSTR, PSTR)], hv)
    pltpu.sync_copy(batchp.at[pl.ds(wid * PSTR, PSTR)], bv)

    def zero(i, c):
        hpl[pl.ds(i * 16, 16)] = jnp.full((16,), -jnp.inf, jnp.float32)
        return c

    lax.fori_loop(0, NG * D // 16, zero, 0)

    def row16(r16, c):
        gv = bv[pl.ds(r16 * 16, 16)]
        for j in range(16):
            gidx = gv[j]
            hrow = hv.at[r16 * 16 + j]
            for cc in range(8):
                off = gidx * D + cc * 16
                cur = hpl[pl.ds(off, 16)]
                hpl[pl.ds(off, 16)] = jnp.maximum(
                    cur, hrow[pl.ds(cc * 16, 16)])
        return c

    lax.fori_loop(0, PSTR // 16, row16, 0)
    pltpu.sync_copy(hpl, pools.at[wid])


def _t3_head(pools_ref, x_ref, batch_ref, shift_ref,
             w0_ref, b0_ref, wl1_ref, bl1_ref, w2_ref, b2_ref, out_ref):
    hi = lax.Precision.HIGHEST
    hp = jnp.max(pools_ref[...], axis=0)

    batch_col = batch_ref[...]
    shift_col = shift_ref[...]
    gid_row = lax.broadcasted_iota(jnp.int32, (1, NG), 1)
    first_ge = (batch_col >= gid_row) & (shift_col < gid_row)
    node_col = lax.broadcasted_iota(jnp.int32, (N, 1), 0)
    overflow = (node_col == N - 1) & (batch_col < gid_row)
    onehot = jnp.where(first_ge | overflow, 1.0, 0.0)
    xr = lax.dot_general(onehot, x_ref[...], (((0,), (0,)), ((), ())),
                         preferred_element_type=jnp.float32, precision=hi)
    news = jnp.maximum(
        jnp.dot(xr, w0_ref[...], preferred_element_type=jnp.float32,
                precision=hi) + b0_ref[...], 0.0)

    cat = jnp.concatenate([news, hp], axis=1)
    h2 = jnp.maximum(
        jnp.dot(cat, wl1_ref[...], preferred_element_type=jnp.float32,
                precision=hi) + bl1_ref[...], 0.0)
    logits = jnp.dot(h2, w2_ref[...], preferred_element_type=jnp.float32,
                     precision=hi) + b2_ref[...]
    mx = jnp.max(logits, axis=1, keepdims=True)
    lse = mx + jnp.log(jnp.sum(jnp.exp(logits - mx), axis=1, keepdims=True))
    out_ref[...] = logits - lse


def kernel(x, edge_index, batch, W1, b1, W0, b0, Wl1, bl1, W2, b2):
    ei0 = edge_index[0].astype(jnp.int32)
    ei1 = edge_index[1].astype(jnp.int32)
    pad = jnp.zeros((GP - E,), jnp.int32)
    srcd = jnp.concatenate([ei0, pad])
    dstd = jnp.concatenate([ei1, pad])

    nrows_g = GP // 128
    codes_f, ticks_f = pl.pallas_call(
        _t0_codes,
        in_specs=[
            pl.BlockSpec((nrows_g, 128), lambda: (0, 0)),
            pl.BlockSpec((nrows_g, 128), lambda: (0, 0)),
        ],
        out_specs=[
            pl.BlockSpec((nrows_g, 128), lambda: (0, 0)),
            pl.BlockSpec((nrows_g, 128), lambda: (0, 0)),
        ],
        out_shape=[
            jax.ShapeDtypeStruct((nrows_g, 128), jnp.int32),
            jax.ShapeDtypeStruct((nrows_g, 128), jnp.int32),
        ],
    )(srcd.reshape(nrows_g, 128), dstd.reshape(nrows_g, 128))
    codes_flat = codes_f.reshape(GP)
    ticks_flat = ticks_f.reshape(GP)

    mesh = plsc.VectorSubcoreMesh(core_axis_name="c", subcore_axis_name="s")

    slot_ref = jax.new_ref(jnp.zeros((TRI + 8,), jnp.int32))
    pl.kernel(
        _s1_scatter,
        out_type=(),
        mesh=mesh,
        scratch_types=[
            pltpu.VMEM((CH,), jnp.int32),
            pltpu.VMEM((CH,), jnp.int32),
            pltpu.SemaphoreType.DMA,
        ],
    )(codes_flat, ticks_flat, slot_ref)
    slot = jax.freeze(slot_ref)

    parts, ga, gb = pl.kernel(
        _s2_degree,
        out_type=(
            jax.ShapeDtypeStruct((NSUB, N), jnp.float32),
            jax.ShapeDtypeStruct((GP,), jnp.int32),
            jax.ShapeDtypeStruct((GP,), jnp.int32),
        ),
        mesh=mesh,
        scratch_types=[
            pltpu.VMEM((CH,), jnp.int32),
            pltpu.VMEM((CH,), jnp.int32),
            pltpu.VMEM((CH,), jnp.int32),
            pltpu.VMEM((CH,), jnp.int32),
            pltpu.VMEM((CH,), jnp.int32),
            pltpu.VMEM((CH,), jnp.int32),
            pltpu.VMEM((N,), jnp.float32),
            pltpu.SemaphoreType.DMA,
        ],
        compiler_params=pltpu.CompilerParams(needs_layout_passes=False),
    )(codes_flat, srcd, dstd, slot)

    g, dinv = pl.pallas_call(
        _t1_prep,
        in_specs=[
            pl.BlockSpec((NSUB, N), lambda: (0, 0)),
            pl.BlockSpec((N, D), lambda: (0, 0)),
            pl.BlockSpec((D, D), lambda: (0, 0)),
        ],
        out_specs=[
            pl.BlockSpec((GG, D), lambda: (0, 0)),
            pl.BlockSpec((N, 1), lambda: (0, 0)),
        ],
        out_shape=[
            jax.ShapeDtypeStruct((GG, D), jnp.float32),
            jax.ShapeDtypeStruct((N, 1), jnp.float32),
        ],
    )(parts, x, W1)

    zrows = jnp.zeros((STRIPE, D), jnp.float32)
    part_out = pl.kernel(
        _s3_spmm,
        out_type=jax.ShapeDtypeStruct((NC, GG, D), jnp.float32),
        mesh=mesh,
        scratch_types=[
            pltpu.VMEM_SHARED((GG, D), jnp.float32),
            pltpu.VMEM((128, D), jnp.float32),
            pltpu.VMEM((128, D), jnp.float32),
            pltpu.VMEM((128,), jnp.int32),
            pltpu.VMEM((128,), jnp.int32),
            pltpu.VMEM((128,), jnp.int32),
            pltpu.VMEM((128,), jnp.int32),
            pltpu.SemaphoreType.DMA,
            pltpu.SemaphoreType.DMA,
            pltpu.SemaphoreType.DMA,
            pltpu.SemaphoreType.DMA,
        ],
    )(ga, gb, g, zrows)

    h1p = pl.pallas_call(
        _t3_h1,
        in_specs=[
            pl.BlockSpec((NC, GG, D), lambda: (0, 0, 0)),
            pl.BlockSpec((GG, D), lambda: (0, 0)),
            pl.BlockSpec((N, 1), lambda: (0, 0)),
            pl.BlockSpec((1, D), lambda: (0, 0)),
        ],
        out_specs=pl.BlockSpec((HP, D), lambda: (0, 0)),
        out_shape=jax.ShapeDtypeStruct((HP, D), jnp.float32),
    )(part_out, g, dinv, b1.reshape(1, D))

    batch_i = batch.astype(jnp.int32)
    batchp = jnp.concatenate(
        [batch_i, jnp.full((HP - N,), NG - 1, jnp.int32)])
    pools = pl.kernel(
        _s4_pool,
        out_type=jax.ShapeDtypeStruct((NSUB, NG * D), jnp.float32),
        mesh=mesh,
        scratch_types=[
            pltpu.VMEM((PSTR, D), jnp.float32),
            pltpu.VMEM((PSTR,), jnp.int32),
            pltpu.VMEM((NG * D,), jnp.float32),
        ],
    )(h1p, batchp)

    out = pl.pallas_call(
        _t3_head,
        in_specs=[
            pl.BlockSpec((NSUB, NG, D), lambda: (0, 0, 0)),
            pl.BlockSpec((N, D), lambda: (0, 0)),
            pl.BlockSpec((N, 1), lambda: (0, 0)),
            pl.BlockSpec((N, 1), lambda: (0, 0)),
            pl.BlockSpec((D, D), lambda: (0, 0)),
            pl.BlockSpec((1, D), lambda: (0, 0)),
            pl.BlockSpec((2 * D, D), lambda: (0, 0)),
            pl.BlockSpec((1, D), lambda: (0, 0)),
            pl.BlockSpec((D, NCLS), lambda: (0, 0)),
            pl.BlockSpec((1, NCLS), lambda: (0, 0)),
        ],
        out_specs=pl.BlockSpec((NG, NCLS), lambda: (0, 0)),
        out_shape=jax.ShapeDtypeStruct((NG, NCLS), jnp.float32),
    )(pools.reshape(NSUB, NG, D), x,
      batch_i.reshape(N, 1),
      jnp.concatenate([jnp.full((1,), -1, jnp.int32),
                       batch_i[:-1]]).reshape(N, 1),
      W0, b0.reshape(1, D), Wl1, bl1.reshape(1, D), W2, b2.reshape(1, NCLS))
    return out


# speedup vs baseline: 1.1940x; 1.0154x over previous
import jax
import jax.numpy as jnp
from jax import lax
from jax.experimental import pallas as pl
from jax.experimental.pallas import tpu as pltpu
from jax.experimental.pallas import tpu_sc as plsc

N = 10000
E = 320000
D = 128
NG = 128
NCLS = 2

TRI = N * (N - 1) // 2
NC, NS = 2, 16
NSUB = NC * NS
ROWS = -(-E // (NSUB * 128))
CH = ROWS * 128
GP = NSUB * CH
TRASH = N
STRIPE = 632
GG = NS * STRIPE
PSTR = 320
HP = NSUB * PSTR
FAST_CID = 0
CH_A = 14720
CH_B = (GP - NS * CH_A) // NS


def _wid():
    return lax.axis_index("s") * NC + lax.axis_index("c")


def _t0_codes(src_ref, dst_ref, code_ref, tick_ref):
    s = src_ref[...]
    d = dst_ref[...]
    lo = jnp.minimum(s, d)
    hi = jnp.maximum(s, d)
    tri = lo * N - (lo * (lo + 1)) // 2 + (hi - lo - 1)
    nrows = GP // 128
    t = (lax.broadcasted_iota(jnp.int32, (nrows, 128), 0) * 128
         + lax.broadcasted_iota(jnp.int32, (nrows, 128), 1))
    loop = s == d
    code_ref[...] = jnp.where(loop, TRI, tri)
    tick_ref[...] = jnp.where(loop, 0, t + 1)


def _s1_scatter(codes_f, ticks_f, slot, idx_a, val_a, idx_b, val_b, sem):
    cid = lax.axis_index("c")
    sid = lax.axis_index("s")

    @pl.when(cid == FAST_CID)
    def _():
        base = sid * CH_A
        pltpu.sync_copy(codes_f.at[pl.ds(base, CH_A)], idx_a)
        pltpu.sync_copy(ticks_f.at[pl.ds(base, CH_A)], val_a)
        pltpu.async_copy(val_a, slot.at[idx_a], sem).wait()

    @pl.when(cid != FAST_CID)
    def _():
        base = NS * CH_A + sid * CH_B
        pltpu.sync_copy(codes_f.at[pl.ds(base, CH_B)], idx_b)
        pltpu.sync_copy(ticks_f.at[pl.ds(base, CH_B)], val_b)
        pltpu.async_copy(val_b, slot.at[idx_b], sem).wait()


def _s2_degree(codes_f, srcd, dstd, slot, parts, ga, gb, idx_v, src_v,
               dst_v, got_v, ga_v, gb_v, deg_v, sem):
    wid = _wid()
    base = wid * CH
    pltpu.sync_copy(codes_f.at[pl.ds(base, CH)], idx_v)
    pltpu.sync_copy(srcd.at[pl.ds(base, CH)], src_v)
    pltpu.sync_copy(dstd.at[pl.ds(base, CH)], dst_v)
    pltpu.async_copy(slot.at[idx_v], got_v, sem).wait()

    def zero(i, c):
        deg_v[pl.ds(i * 16, 16)] = jnp.zeros((16,), jnp.float32)
        return c

    lax.fori_loop(0, N // 16, zero, 0)

    def acc(r, c):
        for cc in range(8):
            off = r * 128 + cc * 16
            got = got_v[pl.ds(off, 16)]
            s = src_v[pl.ds(off, 16)]
            d = dst_v[pl.ds(off, 16)]
            gt = base + off + lax.iota(jnp.int32, 16)
            keepb = got == gt + 1
            keep = jnp.where(keepb, 1.0, 0.0)
            plsc.addupdate_scatter(deg_v, [s], keep)
            plsc.addupdate_scatter(deg_v, [d], keep)
            ga_v[pl.ds(off, 16)] = jnp.where(keepb, s, TRASH)
            gb_v[pl.ds(off, 16)] = jnp.where(keepb, d, TRASH)
        return c

    lax.fori_loop(0, ROWS, acc, 0)
    pltpu.sync_copy(deg_v, parts.at[wid])
    pltpu.sync_copy(ga_v, ga.at[pl.ds(base, CH)])
    pltpu.sync_copy(gb_v, gb.at[pl.ds(base, CH)])


def _t1_prep(parts_ref, x_ref, w1_ref, g_ref, dinv_ref):
    ones32 = jnp.ones((NSUB, 1), jnp.float32)
    deg_col = lax.dot_general(parts_ref[...], ones32,
                              (((0,), (0,)), ((), ())),
                              preferred_element_type=jnp.float32,
                              precision=lax.Precision.HIGHEST) + 1.0
    dinv_col = lax.rsqrt(deg_col)
    h = jnp.dot(x_ref[...], w1_ref[...], preferred_element_type=jnp.float32,
                precision=lax.Precision.HIGHEST)
    g_ref[...] = jnp.concatenate(
        [h * dinv_col, jnp.zeros((GG - N, D), jnp.float32)])
    dinv_ref[...] = dinv_col


def _s3_spmm(ga_f, gb_f, gpad, zrows, part_out, shared, rows0, rows1,
             gi0, si0, gi1, si1, semg, sems0, sems1, semi):
    cid = lax.axis_index("c")
    sid = lax.axis_index("s")
    wid = sid * NC + cid
    base = wid * CH
    pltpu.sync_copy(zrows, shared.at[pl.ds(sid * STRIPE, STRIPE)])
    plsc.subcore_barrier()

    rows = (rows0, rows1)
    gi = (gi0, gi1)
    si = (si0, si1)
    gsrc = (ga_f, gb_f)
    ssrc = (gb_f, ga_f)

    sems = (sems0, sems1)

    def load_and_gather(c, p):
        pltpu.async_copy(gsrc[p].at[pl.ds(base + c * 128, 128)], gi[p], semi)
        pltpu.async_copy(ssrc[p].at[pl.ds(base + c * 128, 128)], si[p], semi)
        pltpu.make_async_copy(gsrc[p].at[pl.ds(base, 128)], gi[p], semi).wait()
        pltpu.make_async_copy(gsrc[p].at[pl.ds(base, 128)], si[p], semi).wait()
        pltpu.async_copy(gpad.at[gi[p]], rows[p], semg)

    def start_scatter(p):
        pltpu.make_async_copy(gpad.at[gi[p]], rows[p], semg).wait()
        pltpu.async_copy(rows[p], shared.at[si[p]], sems[p], add=True)

    def drain_scatter(p):
        pltpu.make_async_copy(rows[p], shared.at[si[p]], sems[p]).wait()

    load_and_gather(0, 0)
    load_and_gather(0, 1)

    def body(i, carry):
        for p in (0, 1):
            start_scatter(p)
            drain_scatter(p)
            load_and_gather(i + 1, p)
        return carry

    lax.fori_loop(0, ROWS - 1, body, 0)
    for p in (0, 1):
        start_scatter(p)
        drain_scatter(p)
    plsc.subcore_barrier()
    pltpu.sync_copy(shared.at[pl.ds(sid * STRIPE, STRIPE)],
                    part_out.at[cid, pl.ds(sid * STRIPE, STRIPE)])


def _t3_h1(p_ref, g_ref, dinv_ref, b1_ref, h1p_ref):
    psum = p_ref[0, :N, :] + p_ref[1, :N, :] + g_ref[:N, :]
    h1 = jnp.maximum(psum * dinv_ref[...] + b1_ref[...], 0.0)
    h1p_ref[...] = jnp.concatenate(
        [h1, jnp.full((HP - N, D), -jnp.inf, jnp.float32)])


def _s4_pool(h1p, batchp, pools, hv, bv, hpl):
    wid = _wid()
    pltpu.sync_copy(h1p.at[pl.ds(wid * PSTR, PSTR)], hv)
    pltpu.sync_copy(batchp.at[pl.ds(wid * PSTR, PSTR)], bv)

    def zero(i, c):
        hpl[pl.ds(i * 16, 16)] = jnp.full((16,), -jnp.inf, jnp.float32)
        return c

    lax.fori_loop(0, NG * D // 16, zero, 0)

    def row16(r16, c):
        gv = bv[pl.ds(r16 * 16, 16)]
        for j in range(16):
            gidx = gv[j]
            hrow = hv.at[r16 * 16 + j]
            for cc in range(8):
                off = gidx * D + cc * 16
                cur = hpl[pl.ds(off, 16)]
                hpl[pl.ds(off, 16)] = jnp.maximum(
                    cur, hrow[pl.ds(cc * 16, 16)])
        return c

    lax.fori_loop(0, PSTR // 16, row16, 0)
    pltpu.sync_copy(hpl, pools.at[wid])


def _t3_head(pools_ref, x_ref, batch_ref, shift_ref,
             w0_ref, b0_ref, wl1_ref, bl1_ref, w2_ref, b2_ref, out_ref):
    hi = lax.Precision.HIGHEST
    hp = jnp.max(pools_ref[...], axis=0)

    batch_col = batch_ref[...]
    shift_col = shift_ref[...]
    gid_row = lax.broadcasted_iota(jnp.int32, (1, NG), 1)
    first_ge = (batch_col >= gid_row) & (shift_col < gid_row)
    node_col = lax.broadcasted_iota(jnp.int32, (N, 1), 0)
    overflow = (node_col == N - 1) & (batch_col < gid_row)
    onehot = jnp.where(first_ge | overflow, 1.0, 0.0)
    xr = lax.dot_general(onehot, x_ref[...], (((0,), (0,)), ((), ())),
                         preferred_element_type=jnp.float32, precision=hi)
    news = jnp.maximum(
        jnp.dot(xr, w0_ref[...], preferred_element_type=jnp.float32,
                precision=hi) + b0_ref[...], 0.0)

    cat = jnp.concatenate([news, hp], axis=1)
    h2 = jnp.maximum(
        jnp.dot(cat, wl1_ref[...], preferred_element_type=jnp.float32,
                precision=hi) + bl1_ref[...], 0.0)
    logits = jnp.dot(h2, w2_ref[...], preferred_element_type=jnp.float32,
                     precision=hi) + b2_ref[...]
    mx = jnp.max(logits, axis=1, keepdims=True)
    lse = mx + jnp.log(jnp.sum(jnp.exp(logits - mx), axis=1, keepdims=True))
    out_ref[...] = logits - lse


def kernel(x, edge_index, batch, W1, b1, W0, b0, Wl1, bl1, W2, b2):
    ei0 = edge_index[0].astype(jnp.int32)
    ei1 = edge_index[1].astype(jnp.int32)
    pad = jnp.zeros((GP - E,), jnp.int32)
    srcd = jnp.concatenate([ei0, pad])
    dstd = jnp.concatenate([ei1, pad])

    nrows_g = GP // 128
    codes_f, ticks_f = pl.pallas_call(
        _t0_codes,
        in_specs=[
            pl.BlockSpec((nrows_g, 128), lambda: (0, 0)),
            pl.BlockSpec((nrows_g, 128), lambda: (0, 0)),
        ],
        out_specs=[
            pl.BlockSpec((nrows_g, 128), lambda: (0, 0)),
            pl.BlockSpec((nrows_g, 128), lambda: (0, 0)),
        ],
        out_shape=[
            jax.ShapeDtypeStruct((nrows_g, 128), jnp.int32),
            jax.ShapeDtypeStruct((nrows_g, 128), jnp.int32),
        ],
    )(srcd.reshape(nrows_g, 128), dstd.reshape(nrows_g, 128))
    codes_flat = codes_f.reshape(GP)
    ticks_flat = ticks_f.reshape(GP)

    mesh = plsc.VectorSubcoreMesh(core_axis_name="c", subcore_axis_name="s")

    slot_ref = jax.new_ref(jnp.zeros((TRI + 8,), jnp.int32))
    pl.kernel(
        _s1_scatter,
        out_type=(),
        mesh=mesh,
        scratch_types=[
            pltpu.VMEM((CH_A,), jnp.int32),
            pltpu.VMEM((CH_A,), jnp.int32),
            pltpu.VMEM((CH_B,), jnp.int32),
            pltpu.VMEM((CH_B,), jnp.int32),
            pltpu.SemaphoreType.DMA,
        ],
    )(codes_flat, ticks_flat, slot_ref)
    slot = jax.freeze(slot_ref)

    parts, ga, gb = pl.kernel(
        _s2_degree,
        out_type=(
            jax.ShapeDtypeStruct((NSUB, N), jnp.float32),
            jax.ShapeDtypeStruct((GP,), jnp.int32),
            jax.ShapeDtypeStruct((GP,), jnp.int32),
        ),
        mesh=mesh,
        scratch_types=[
            pltpu.VMEM((CH,), jnp.int32),
            pltpu.VMEM((CH,), jnp.int32),
            pltpu.VMEM((CH,), jnp.int32),
            pltpu.VMEM((CH,), jnp.int32),
            pltpu.VMEM((CH,), jnp.int32),
            pltpu.VMEM((CH,), jnp.int32),
            pltpu.VMEM((N,), jnp.float32),
            pltpu.SemaphoreType.DMA,
        ],
        compiler_params=pltpu.CompilerParams(needs_layout_passes=False),
    )(codes_flat, srcd, dstd, slot)

    g, dinv = pl.pallas_call(
        _t1_prep,
        in_specs=[
            pl.BlockSpec((NSUB, N), lambda: (0, 0)),
            pl.BlockSpec((N, D), lambda: (0, 0)),
            pl.BlockSpec((D, D), lambda: (0, 0)),
        ],
        out_specs=[
            pl.BlockSpec((GG, D), lambda: (0, 0)),
            pl.BlockSpec((N, 1), lambda: (0, 0)),
        ],
        out_shape=[
            jax.ShapeDtypeStruct((GG, D), jnp.float32),
            jax.ShapeDtypeStruct((N, 1), jnp.float32),
        ],
    )(parts, x, W1)

    zrows = jnp.zeros((STRIPE, D), jnp.float32)
    part_out = pl.kernel(
        _s3_spmm,
        out_type=jax.ShapeDtypeStruct((NC, GG, D), jnp.float32),
        mesh=mesh,
        scratch_types=[
            pltpu.VMEM_SHARED((GG, D), jnp.float32),
            pltpu.VMEM((128, D), jnp.float32),
            pltpu.VMEM((128, D), jnp.float32),
            pltpu.VMEM((128,), jnp.int32),
            pltpu.VMEM((128,), jnp.int32),
            pltpu.VMEM((128,), jnp.int32),
            pltpu.VMEM((128,), jnp.int32),
            pltpu.SemaphoreType.DMA,
            pltpu.SemaphoreType.DMA,
            pltpu.SemaphoreType.DMA,
            pltpu.SemaphoreType.DMA,
        ],
    )(ga, gb, g, zrows)

    h1p = pl.pallas_call(
        _t3_h1,
        in_specs=[
            pl.BlockSpec((NC, GG, D), lambda: (0, 0, 0)),
            pl.BlockSpec((GG, D), lambda: (0, 0)),
            pl.BlockSpec((N, 1), lambda: (0, 0)),
            pl.BlockSpec((1, D), lambda: (0, 0)),
        ],
        out_specs=pl.BlockSpec((HP, D), lambda: (0, 0)),
        out_shape=jax.ShapeDtypeStruct((HP, D), jnp.float32),
    )(part_out, g, dinv, b1.reshape(1, D))

    batch_i = batch.astype(jnp.int32)
    batchp = jnp.concatenate(
        [batch_i, jnp.full((HP - N,), NG - 1, jnp.int32)])
    pools = pl.kernel(
        _s4_pool,
        out_type=jax.ShapeDtypeStruct((NSUB, NG * D), jnp.float32),
        mesh=mesh,
        scratch_types=[
            pltpu.VMEM((PSTR, D), jnp.float32),
            pltpu.VMEM((PSTR,), jnp.int32),
            pltpu.VMEM((NG * D,), jnp.float32),
        ],
    )(h1p, batchp)

    out = pl.pallas_call(
        _t3_head,
        in_specs=[
            pl.BlockSpec((NSUB, NG, D), lambda: (0, 0, 0)),
            pl.BlockSpec((N, D), lambda: (0, 0)),
            pl.BlockSpec((N, 1), lambda: (0, 0)),
            pl.BlockSpec((N, 1), lambda: (0, 0)),
            pl.BlockSpec((D, D), lambda: (0, 0)),
            pl.BlockSpec((1, D), lambda: (0, 0)),
            pl.BlockSpec((2 * D, D), lambda: (0, 0)),
            pl.BlockSpec((1, D), lambda: (0, 0)),
            pl.BlockSpec((D, NCLS), lambda: (0, 0)),
            pl.BlockSpec((1, NCLS), lambda: (0, 0)),
        ],
        out_specs=pl.BlockSpec((NG, NCLS), lambda: (0, 0)),
        out_shape=jax.ShapeDtypeStruct((NG, NCLS), jnp.float32),
    )(pools.reshape(NSUB, NG, D), x,
      batch_i.reshape(N, 1),
      jnp.concatenate([jnp.full((1,), -1, jnp.int32),
                       batch_i[:-1]]).reshape(N, 1),
      W0, b0.reshape(1, D), Wl1, bl1.reshape(1, D), W2, b2.reshape(1, NCLS))
    return out


# speedup vs baseline: 1.1968x; 1.0024x over previous
import jax
import jax.numpy as jnp
from jax import lax
from jax.experimental import pallas as pl
from jax.experimental.pallas import tpu as pltpu
from jax.experimental.pallas import tpu_sc as plsc

N = 10000
E = 320000
D = 128
NG = 128
NCLS = 2

TRI = N * (N - 1) // 2
NC, NS = 2, 16
NSUB = NC * NS
ROWS = -(-E // (NSUB * 128))
CH = ROWS * 128
GP = NSUB * CH
TRASH = N
STRIPE = 632
GG = NS * STRIPE
PSTR = 320
HP = NSUB * PSTR
FAST_CID = 0
CH_A = 14720
CH_B = (GP - NS * CH_A) // NS
NCH_A = 106
NCH_B = (GP // 128 - NS * NCH_A) // NS


def _wid():
    return lax.axis_index("s") * NC + lax.axis_index("c")


def _t0_codes(src_ref, dst_ref, code_ref, tick_ref):
    s = src_ref[...]
    d = dst_ref[...]
    lo = jnp.minimum(s, d)
    hi = jnp.maximum(s, d)
    tri = lo * N - (lo * (lo + 1)) // 2 + (hi - lo - 1)
    nrows = GP // 128
    t = (lax.broadcasted_iota(jnp.int32, (nrows, 128), 0) * 128
         + lax.broadcasted_iota(jnp.int32, (nrows, 128), 1))
    loop = s == d
    code_ref[...] = jnp.where(loop, TRI, tri)
    tick_ref[...] = jnp.where(loop, 0, t + 1)


def _s1_scatter(codes_f, ticks_f, slot, idx_a, val_a, idx_b, val_b, sem):
    cid = lax.axis_index("c")
    sid = lax.axis_index("s")

    @pl.when(cid == FAST_CID)
    def _():
        base = sid * CH_A
        pltpu.sync_copy(codes_f.at[pl.ds(base, CH_A)], idx_a)
        pltpu.sync_copy(ticks_f.at[pl.ds(base, CH_A)], val_a)
        pltpu.async_copy(val_a, slot.at[idx_a], sem).wait()

    @pl.when(cid != FAST_CID)
    def _():
        base = NS * CH_A + sid * CH_B
        pltpu.sync_copy(codes_f.at[pl.ds(base, CH_B)], idx_b)
        pltpu.sync_copy(ticks_f.at[pl.ds(base, CH_B)], val_b)
        pltpu.async_copy(val_b, slot.at[idx_b], sem).wait()


def _s2_degree(codes_f, srcd, dstd, slot, parts, ga, gb, idx_v, src_v,
               dst_v, got_v, ga_v, gb_v, deg_v, sem):
    wid = _wid()
    base = wid * CH
    pltpu.sync_copy(codes_f.at[pl.ds(base, CH)], idx_v)
    pltpu.sync_copy(srcd.at[pl.ds(base, CH)], src_v)
    pltpu.sync_copy(dstd.at[pl.ds(base, CH)], dst_v)
    pltpu.async_copy(slot.at[idx_v], got_v, sem).wait()

    def zero(i, c):
        deg_v[pl.ds(i * 16, 16)] = jnp.zeros((16,), jnp.float32)
        return c

    lax.fori_loop(0, N // 16, zero, 0)

    def acc(r, c):
        for cc in range(8):
            off = r * 128 + cc * 16
            got = got_v[pl.ds(off, 16)]
            s = src_v[pl.ds(off, 16)]
            d = dst_v[pl.ds(off, 16)]
            gt = base + off + lax.iota(jnp.int32, 16)
            keepb = got == gt + 1
            keep = jnp.where(keepb, 1.0, 0.0)
            plsc.addupdate_scatter(deg_v, [s], keep)
            plsc.addupdate_scatter(deg_v, [d], keep)
            ga_v[pl.ds(off, 16)] = jnp.where(keepb, s, TRASH)
            gb_v[pl.ds(off, 16)] = jnp.where(keepb, d, TRASH)
        return c

    lax.fori_loop(0, ROWS, acc, 0)
    pltpu.sync_copy(deg_v, parts.at[wid])
    pltpu.sync_copy(ga_v, ga.at[pl.ds(base, CH)])
    pltpu.sync_copy(gb_v, gb.at[pl.ds(base, CH)])


def _t1_prep(parts_ref, x_ref, w1_ref, g_ref, dinv_ref):
    ones32 = jnp.ones((NSUB, 1), jnp.float32)
    deg_col = lax.dot_general(parts_ref[...], ones32,
                              (((0,), (0,)), ((), ())),
                              preferred_element_type=jnp.float32,
                              precision=lax.Precision.HIGHEST) + 1.0
    dinv_col = lax.rsqrt(deg_col)
    h = jnp.dot(x_ref[...], w1_ref[...], preferred_element_type=jnp.float32,
                precision=lax.Precision.HIGHEST)
    g_ref[...] = jnp.concatenate(
        [h * dinv_col, jnp.zeros((GG - N, D), jnp.float32)])
    dinv_ref[...] = dinv_col


def _s3_spmm(ga_f, gb_f, gpad, zrows, part_out, shared, rows0, rows1,
             gi0, si0, gi1, si1, semg, sems0, sems1, semi):
    cid = lax.axis_index("c")
    sid = lax.axis_index("s")
    pltpu.sync_copy(zrows, shared.at[pl.ds(sid * STRIPE, STRIPE)])
    plsc.subcore_barrier()

    rows = (rows0, rows1)
    gi = (gi0, gi1)
    si = (si0, si1)
    gsrc = (ga_f, gb_f)
    ssrc = (gb_f, ga_f)

    sems = (sems0, sems1)

    def run(cbase, nch):
        def load_and_gather(c, p):
            off = (cbase + c) * 128
            pltpu.async_copy(gsrc[p].at[pl.ds(off, 128)], gi[p], semi)
            pltpu.async_copy(ssrc[p].at[pl.ds(off, 128)], si[p], semi)
            pltpu.make_async_copy(gsrc[p].at[pl.ds(0, 128)], gi[p],
                                  semi).wait()
            pltpu.make_async_copy(gsrc[p].at[pl.ds(0, 128)], si[p],
                                  semi).wait()
            pltpu.async_copy(gpad.at[gi[p]], rows[p], semg)

        def start_scatter(p):
            pltpu.make_async_copy(gpad.at[gi[p]], rows[p], semg).wait()
            pltpu.async_copy(rows[p], shared.at[si[p]], sems[p], add=True)

        def drain_scatter(p):
            pltpu.make_async_copy(rows[p], shared.at[si[p]], sems[p]).wait()

        load_and_gather(0, 0)
        load_and_gather(0, 1)

        def body(i, carry):
            for p in (0, 1):
                start_scatter(p)
                drain_scatter(p)
                load_and_gather(i + 1, p)
            return carry

        lax.fori_loop(0, nch - 1, body, 0)
        for p in (0, 1):
            start_scatter(p)
            drain_scatter(p)

    @pl.when(cid == FAST_CID)
    def _():
        run(sid * NCH_A, NCH_A)

    @pl.when(cid != FAST_CID)
    def _():
        run(NS * NCH_A + sid * NCH_B, NCH_B)
    plsc.subcore_barrier()
    pltpu.sync_copy(shared.at[pl.ds(sid * STRIPE, STRIPE)],
                    part_out.at[cid, pl.ds(sid * STRIPE, STRIPE)])


def _t3_h1(p_ref, g_ref, dinv_ref, b1_ref, h1p_ref):
    psum = p_ref[0, :N, :] + p_ref[1, :N, :] + g_ref[:N, :]
    h1 = jnp.maximum(psum * dinv_ref[...] + b1_ref[...], 0.0)
    h1p_ref[...] = jnp.concatenate(
        [h1, jnp.full((HP - N, D), -jnp.inf, jnp.float32)])


def _s4_pool(h1p, batchp, pools, hv, bv, hpl):
    wid = _wid()
    pltpu.sync_copy(h1p.at[pl.ds(wid * PSTR, PSTR)], hv)
    pltpu.sync_copy(batchp.at[pl.ds(wid * PSTR, PSTR)], bv)

    def zero(i, c):
        hpl[pl.ds(i * 16, 16)] = jnp.full((16,), -jnp.inf, jnp.float32)
        return c

    lax.fori_loop(0, NG * D // 16, zero, 0)

    def row16(r16, c):
        gv = bv[pl.ds(r16 * 16, 16)]
        for j in range(16):
            gidx = gv[j]
            hrow = hv.at[r16 * 16 + j]
            for cc in range(8):
                off = gidx * D + cc * 16
                cur = hpl[pl.ds(off, 16)]
                hpl[pl.ds(off, 16)] = jnp.maximum(
                    cur, hrow[pl.ds(cc * 16, 16)])
        return c

    lax.fori_loop(0, PSTR // 16, row16, 0)
    pltpu.sync_copy(hpl, pools.at[wid])


def _t3_head(pools_ref, x_ref, batch_ref, shift_ref,
             w0_ref, b0_ref, wl1_ref, bl1_ref, w2_ref, b2_ref, out_ref):
    hi = lax.Precision.HIGHEST
    hp = jnp.max(pools_ref[...], axis=0)

    batch_col = batch_ref[...]
    shift_col = shift_ref[...]
    gid_row = lax.broadcasted_iota(jnp.int32, (1, NG), 1)
    first_ge = (batch_col >= gid_row) & (shift_col < gid_row)
    node_col = lax.broadcasted_iota(jnp.int32, (N, 1), 0)
    overflow = (node_col == N - 1) & (batch_col < gid_row)
    onehot = jnp.where(first_ge | overflow, 1.0, 0.0)
    xr = lax.dot_general(onehot, x_ref[...], (((0,), (0,)), ((), ())),
                         preferred_element_type=jnp.float32, precision=hi)
    news = jnp.maximum(
        jnp.dot(xr, w0_ref[...], preferred_element_type=jnp.float32,
                precision=hi) + b0_ref[...], 0.0)

    cat = jnp.concatenate([news, hp], axis=1)
    h2 = jnp.maximum(
        jnp.dot(cat, wl1_ref[...], preferred_element_type=jnp.float32,
                precision=hi) + bl1_ref[...], 0.0)
    logits = jnp.dot(h2, w2_ref[...], preferred_element_type=jnp.float32,
                     precision=hi) + b2_ref[...]
    mx = jnp.max(logits, axis=1, keepdims=True)
    lse = mx + jnp.log(jnp.sum(jnp.exp(logits - mx), axis=1, keepdims=True))
    out_ref[...] = logits - lse


def kernel(x, edge_index, batch, W1, b1, W0, b0, Wl1, bl1, W2, b2):
    ei0 = edge_index[0].astype(jnp.int32)
    ei1 = edge_index[1].astype(jnp.int32)
    pad = jnp.zeros((GP - E,), jnp.int32)
    srcd = jnp.concatenate([ei0, pad])
    dstd = jnp.concatenate([ei1, pad])

    nrows_g = GP // 128
    codes_f, ticks_f = pl.pallas_call(
        _t0_codes,
        in_specs=[
            pl.BlockSpec((nrows_g, 128), lambda: (0, 0)),
            pl.BlockSpec((nrows_g, 128), lambda: (0, 0)),
        ],
        out_specs=[
            pl.BlockSpec((nrows_g, 128), lambda: (0, 0)),
            pl.BlockSpec((nrows_g, 128), lambda: (0, 0)),
        ],
        out_shape=[
            jax.ShapeDtypeStruct((nrows_g, 128), jnp.int32),
            jax.ShapeDtypeStruct((nrows_g, 128), jnp.int32),
        ],
    )(srcd.reshape(nrows_g, 128), dstd.reshape(nrows_g, 128))
    codes_flat = codes_f.reshape(GP)
    ticks_flat = ticks_f.reshape(GP)

    mesh = plsc.VectorSubcoreMesh(core_axis_name="c", subcore_axis_name="s")

    slot_ref = jax.new_ref(jnp.zeros((TRI + 8,), jnp.int32))
    pl.kernel(
        _s1_scatter,
        out_type=(),
        mesh=mesh,
        scratch_types=[
            pltpu.VMEM((CH_A,), jnp.int32),
            pltpu.VMEM((CH_A,), jnp.int32),
            pltpu.VMEM((CH_B,), jnp.int32),
            pltpu.VMEM((CH_B,), jnp.int32),
            pltpu.SemaphoreType.DMA,
        ],
    )(codes_flat, ticks_flat, slot_ref)
    slot = jax.freeze(slot_ref)

    parts, ga, gb = pl.kernel(
        _s2_degree,
        out_type=(
            jax.ShapeDtypeStruct((NSUB, N), jnp.float32),
            jax.ShapeDtypeStruct((GP,), jnp.int32),
            jax.ShapeDtypeStruct((GP,), jnp.int32),
        ),
        mesh=mesh,
        scratch_types=[
            pltpu.VMEM((CH,), jnp.int32),
            pltpu.VMEM((CH,), jnp.int32),
            pltpu.VMEM((CH,), jnp.int32),
            pltpu.VMEM((CH,), jnp.int32),
            pltpu.VMEM((CH,), jnp.int32),
            pltpu.VMEM((CH,), jnp.int32),
            pltpu.VMEM((N,), jnp.float32),
            pltpu.SemaphoreType.DMA,
        ],
        compiler_params=pltpu.CompilerParams(needs_layout_passes=False),
    )(codes_flat, srcd, dstd, slot)

    g, dinv = pl.pallas_call(
        _t1_prep,
        in_specs=[
            pl.BlockSpec((NSUB, N), lambda: (0, 0)),
            pl.BlockSpec((N, D), lambda: (0, 0)),
            pl.BlockSpec((D, D), lambda: (0, 0)),
        ],
        out_specs=[
            pl.BlockSpec((GG, D), lambda: (0, 0)),
            pl.BlockSpec((N, 1), lambda: (0, 0)),
        ],
        out_shape=[
            jax.ShapeDtypeStruct((GG, D), jnp.float32),
            jax.ShapeDtypeStruct((N, 1), jnp.float32),
        ],
    )(parts, x, W1)

    zrows = jnp.zeros((STRIPE, D), jnp.float32)
    part_out = pl.kernel(
        _s3_spmm,
        out_type=jax.ShapeDtypeStruct((NC, GG, D), jnp.float32),
        mesh=mesh,
        scratch_types=[
            pltpu.VMEM_SHARED((GG, D), jnp.float32),
            pltpu.VMEM((128, D), jnp.float32),
            pltpu.VMEM((128, D), jnp.float32),
            pltpu.VMEM((128,), jnp.int32),
            pltpu.VMEM((128,), jnp.int32),
            pltpu.VMEM((128,), jnp.int32),
            pltpu.VMEM((128,), jnp.int32),
            pltpu.SemaphoreType.DMA,
            pltpu.SemaphoreType.DMA,
            pltpu.SemaphoreType.DMA,
            pltpu.SemaphoreType.DMA,
        ],
    )(ga, gb, g, zrows)

    h1p = pl.pallas_call(
        _t3_h1,
        in_specs=[
            pl.BlockSpec((NC, GG, D), lambda: (0, 0, 0)),
            pl.BlockSpec((GG, D), lambda: (0, 0)),
            pl.BlockSpec((N, 1), lambda: (0, 0)),
            pl.BlockSpec((1, D), lambda: (0, 0)),
        ],
        out_specs=pl.BlockSpec((HP, D), lambda: (0, 0)),
        out_shape=jax.ShapeDtypeStruct((HP, D), jnp.float32),
    )(part_out, g, dinv, b1.reshape(1, D))

    batch_i = batch.astype(jnp.int32)
    batchp = jnp.concatenate(
        [batch_i, jnp.full((HP - N,), NG - 1, jnp.int32)])
    pools = pl.kernel(
        _s4_pool,
        out_type=jax.ShapeDtypeStruct((NSUB, NG * D), jnp.float32),
        mesh=mesh,
        scratch_types=[
            pltpu.VMEM((PSTR, D), jnp.float32),
            pltpu.VMEM((PSTR,), jnp.int32),
            pltpu.VMEM((NG * D,), jnp.float32),
        ],
    )(h1p, batchp)

    out = pl.pallas_call(
        _t3_head,
        in_specs=[
            pl.BlockSpec((NSUB, NG, D), lambda: (0, 0, 0)),
            pl.BlockSpec((N, D), lambda: (0, 0)),
            pl.BlockSpec((N, 1), lambda: (0, 0)),
            pl.BlockSpec((N, 1), lambda: (0, 0)),
            pl.BlockSpec((D, D), lambda: (0, 0)),
            pl.BlockSpec((1, D), lambda: (0, 0)),
            pl.BlockSpec((2 * D, D), lambda: (0, 0)),
            pl.BlockSpec((1, D), lambda: (0, 0)),
            pl.BlockSpec((D, NCLS), lambda: (0, 0)),
            pl.BlockSpec((1, NCLS), lambda: (0, 0)),
        ],
        out_specs=pl.BlockSpec((NG, NCLS), lambda: (0, 0)),
        out_shape=jax.ShapeDtypeStruct((NG, NCLS), jnp.float32),
    )(pools.reshape(NSUB, NG, D), x,
      batch_i.reshape(N, 1),
      jnp.concatenate([jnp.full((1,), -1, jnp.int32),
                       batch_i[:-1]]).reshape(N, 1),
      W0, b0.reshape(1, D), Wl1, bl1.reshape(1, D), W2, b2.reshape(1, NCLS))
    return out
